# baseline (speedup 1.0000x reference)
import functools

import jax
import jax.numpy as jnp
from jax import lax
from jax.experimental import pallas as pl
from jax.experimental.pallas import tpu as pltpu
from jax.experimental.pallas import tpu_sc as plsc

_BB = 64

_NC = 2
_NS = 16
_NW = _NC * _NS
_L = 16


def _caps_body(x_ref, dist_ref, feat_ref):
    xb = x_ref[...]
    BB, C, D = xb.shape
    sumsq = jnp.sum(xb * xb, axis=1)
    norm = jnp.sqrt(sumsq)
    mx = jnp.max(norm, axis=1, keepdims=True)
    e = jnp.exp(norm - mx)
    dist_ref[...] = e / jnp.sum(e, axis=1, keepdims=True)
    d_iota = jax.lax.broadcasted_iota(jnp.int32, norm.shape, 1)
    idx = jnp.min(jnp.where(norm == mx, d_iota, D), axis=1,
                  keepdims=True)
    mask = d_iota == idx
    masked = jnp.where(mask[:, None, :], xb, 0.0)
    feat_ref[...] = masked.reshape(BB, C * D)


def _feat_body(x_ref, feat_ref):
    xb = x_ref[...]
    BB, C, D = xb.shape
    sumsq = jnp.sum(xb * xb, axis=1)
    mx = jnp.max(sumsq, axis=1, keepdims=True)
    d_iota = jax.lax.broadcasted_iota(jnp.int32, sumsq.shape, 1)
    idx = jnp.min(jnp.where(sumsq == mx, d_iota, D), axis=1,
                  keepdims=True)
    mask = d_iota == idx
    masked = jnp.where(mask[:, None, :], xb, 0.0)
    feat_ref[...] = masked.reshape(BB, C * D)


def _tc_feat_kernel(x):
    B, C, D = x.shape
    return pl.pallas_call(
        _feat_body,
        grid=(B // _BB,),
        in_specs=[pl.BlockSpec((_BB, C, D), lambda i: (i, 0, 0))],
        out_specs=pl.BlockSpec((_BB, C * D), lambda i: (i, 0)),
        out_shape=jax.ShapeDtypeStruct((B, C * D), x.dtype),
    )(x)


def _tc_kernel(x):
    B, C, D = x.shape
    return pl.pallas_call(
        _caps_body,
        grid=(B // _BB,),
        in_specs=[pl.BlockSpec((_BB, C, D), lambda i: (i, 0, 0))],
        out_specs=[
            pl.BlockSpec((_BB, D), lambda i: (i, 0)),
            pl.BlockSpec((_BB, C * D), lambda i: (i, 0)),
        ],
        out_shape=[
            jax.ShapeDtypeStruct((B, D), x.dtype),
            jax.ShapeDtypeStruct((B, C * D), x.dtype),
        ],
    )(x)


def _sqrt16(s):
    i = lax.bitcast_convert_type(s, jnp.int32)
    r = lax.bitcast_convert_type(jnp.int32(0x5F3759DF) - (i >> 1), jnp.float32)
    for _ in range(3):
        r = r * (1.5 - 0.5 * s * r * r)
    return s * r


def _rot16(v, lane, sh):
    idx = (lane + sh) & (_L - 1)
    dnums = lax.GatherDimensionNumbers(
        offset_dims=(), collapsed_slice_dims=(0,), start_index_map=(0,))
    return lax.gather(v, idx[:, None], dnums, slice_sizes=(1,),
                      mode=lax.GatherScatterMode.PROMISE_IN_BOUNDS)


def _alltree16(v, lane, op):
    for sh in (8, 4, 2, 1):
        v = op(v, _rot16(v, lane, sh))
    return v


def _sc_kernel(x):
    B, C, D = x.shape
    CD = C * D
    spw = B // _NW
    nch = D // _L

    mesh = plsc.VectorSubcoreMesh(core_axis_name="c", subcore_axis_name="s")

    @functools.partial(
        pl.kernel,
        mesh=mesh,
        out_type=jax.ShapeDtypeStruct((B, D), jnp.float32),
        scratch_types=[
            pltpu.VMEM((C, D), jnp.float32),
            pltpu.VMEM((C, D), jnp.float32),
            pltpu.VMEM((D,), jnp.float32),
            pltpu.SemaphoreType.DMA,
            pltpu.SemaphoreType.DMA,
        ],
    )
    def sc_caps(x_hbm, dist_hbm, x_v0, x_v1, d_v, si0, si1):
        wid = lax.axis_index("s") * _NC + lax.axis_index("c")
        b0 = wid * spw
        lane = lax.iota(jnp.int32, _L)
        zero16 = jnp.zeros((_L,), jnp.float32)

        def process(t, j, x_v, prev_base):
            def csum(c, accs):
                out = []
                for k in range(nch):
                    v = x_v[c, pl.ds(k * _L, _L)]
                    out.append(accs[k] + v * v)
                return tuple(out)
            accs = lax.fori_loop(
                0, C, csum, tuple(zero16 for _ in range(nch)))

            norms = [_sqrt16(accs[k]) for k in range(nch)]
            nmax16 = norms[0]
            for k in range(1, nch):
                nmax16 = jnp.maximum(nmax16, norms[k])
            nmax = _alltree16(nmax16, lane, jnp.maximum)
            es = [jnp.exp(norms[k] - nmax) for k in range(nch)]
            s16 = es[0]
            for k in range(1, nch):
                s16 = s16 + es[k]
            inv = 1.0 / _alltree16(s16, lane, jnp.add)
            for k in range(nch):
                d_v[pl.ds(k * _L, _L)] = es[k] * inv
            pltpu.sync_copy(d_v, dist_hbm.at[b0 + t])
            return prev_base

        pltpu.async_copy(x_hbm.at[b0], x_v0, si0)

        def body2(j, carry):
            pb0, pb1 = carry
            t0 = 2 * j
            pltpu.async_copy(x_hbm.at[b0 + t0 + 1], x_v1, si1)
            pltpu.make_async_copy(x_hbm.at[b0], x_v0, si0).wait()
            nb0 = process(t0, j, x_v0, pb0)

            @pl.when(j < spw // 2 - 1)
            def _():
                pltpu.async_copy(x_hbm.at[b0 + t0 + 2], x_v0, si0)
            pltpu.make_async_copy(x_hbm.at[b0], x_v1, si1).wait()
            nb1 = process(t0 + 1, j, x_v1, nb0)
            return nb1, nb1

        lax.fori_loop(0, spw // 2, body2, (jnp.int32(0), jnp.int32(0)))

    return sc_caps(x)


def kernel(x):
    dist = _sc_kernel(x)
    feat = _tc_feat_kernel(x)
    return dist, feat


# speedup vs baseline: 1.6624x; 1.6624x over previous
import jax
import jax.numpy as jnp
from jax.experimental import pallas as pl

_BB = 64


def _caps_body(x_ref, dist_ref, feat_ref):
    xb = x_ref[...]
    BB, C, D = xb.shape
    sumsq = jnp.sum(xb * xb, axis=1)
    norm = jnp.sqrt(sumsq)
    mx = jnp.max(norm, axis=1, keepdims=True)
    e = jnp.exp(norm - mx)
    dist_ref[...] = e / jnp.sum(e, axis=1, keepdims=True)
    d_iota = jax.lax.broadcasted_iota(jnp.int32, norm.shape, 1)
    idx = jnp.min(jnp.where(norm == mx, d_iota, D), axis=1,
                  keepdims=True)
    mask = d_iota == idx
    masked = jnp.where(mask[:, None, :], xb, 0.0)
    feat_ref[...] = masked.reshape(BB, C * D)


def kernel(x):
    B, C, D = x.shape
    dist, feat = pl.pallas_call(
        _caps_body,
        grid=(B // _BB,),
        in_specs=[pl.BlockSpec((_BB, C, D), lambda i: (i, 0, 0))],
        out_specs=[
            pl.BlockSpec((_BB, D), lambda i: (i, 0)),
            pl.BlockSpec((_BB, C * D), lambda i: (i, 0)),
        ],
        out_shape=[
            jax.ShapeDtypeStruct((B, D), x.dtype),
            jax.ShapeDtypeStruct((B, C * D), x.dtype),
        ],
    )(x)
    return dist, feat
